# baseline (speedup 1.0000x reference)
import jax
import jax.numpy as jnp
from jax import lax
from jax.experimental import pallas as pl
from jax.experimental.pallas import tpu as pltpu
from jax.experimental.pallas import tpu_sc as plsc

NUM_CLAUSES_P1 = 9
HIDDEN = 2048
LANES = 16
BATCH = 16384

_INFO = plsc.get_sparse_core_info()
NC = _INFO.num_cores
NS = _INFO.num_subcores
NW = NC * NS
B_PER_W = BATCH // NW
NGROUPS = B_PER_W // LANES


def _sc_body(idx_hbm, table_hbm, out_hbm, idx_s, sem):
    cid = lax.axis_index("c")
    sid = lax.axis_index("s")
    wid = sid * NC + cid
    base = wid * B_PER_W

    pltpu.sync_copy(idx_hbm.at[pl.ds(base, B_PER_W)], idx_s)

    def group_step(rg, carry):
        ivec = idx_s[pl.ds(rg * LANES, LANES)] * HIDDEN
        row0 = (base + rg * LANES) * HIDDEN
        for l in range(LANES):
            pltpu.make_async_copy(
                table_hbm.at[pl.ds(pl.multiple_of(ivec[l], HIDDEN), HIDDEN)],
                out_hbm.at[pl.ds(pl.multiple_of(row0 + l * HIDDEN, HIDDEN),
                                 HIDDEN)],
                sem,
            ).start()
        return carry

    lax.fori_loop(0, NGROUPS, group_step, 0)

    def drain_step(r, carry):
        pltpu.make_async_copy(
            table_hbm.at[pl.ds(0, HIDDEN)],
            out_hbm.at[pl.ds(pl.multiple_of(base * HIDDEN, HIDDEN), HIDDEN)],
            sem,
        ).wait()
        return carry

    lax.fori_loop(0, B_PER_W, drain_step, 0)


@jax.jit
def kernel(clause_indices, embeddings):
    idx = clause_indices.astype(jnp.int32)
    table_flat = embeddings.reshape(NUM_CLAUSES_P1 * HIDDEN)
    mesh = plsc.VectorSubcoreMesh(core_axis_name="c", subcore_axis_name="s")
    f = pl.kernel(
        _sc_body,
        out_type=jax.ShapeDtypeStruct((BATCH * HIDDEN,), jnp.float32),
        mesh=mesh,
        compiler_params=pltpu.CompilerParams(needs_layout_passes=False),
        scratch_types=[
            pltpu.VMEM((B_PER_W,), jnp.int32),
            pltpu.SemaphoreType.DMA,
        ],
    )
    return f(idx, table_flat).reshape(BATCH, HIDDEN)


# speedup vs baseline: 20.1051x; 20.1051x over previous
import jax
import jax.numpy as jnp
from jax import lax
from jax.experimental import pallas as pl
from jax.experimental.pallas import tpu as pltpu
from jax.experimental.pallas import tpu_sc as plsc

NUM_CLAUSES_P1 = 9
HIDDEN = 2048
LANES = 16
NGRP = HIDDEN // LANES
BATCH = 16384

_INFO = plsc.get_sparse_core_info()
NC = _INFO.num_cores
NS = _INFO.num_subcores
NW = NC * NS
B_PER_W = BATCH // NW
CHUNK = 16
NCHUNK = B_PER_W // CHUNK
NBUF = 2
LBATCH = 8


def _sc_body(idx_hbm, table_hbm, out_hbm, table_v, idx_s,
             buf0, buf1, ss0, ss1):
    bufs = (buf0, buf1)
    ssems = (ss0, ss1)
    cid = lax.axis_index("c")
    sid = lax.axis_index("s")
    wid = sid * NC + cid
    base = wid * B_PER_W

    pltpu.sync_copy(table_hbm, table_v)
    pltpu.sync_copy(idx_hbm.at[pl.ds(base, B_PER_W)], idx_s)

    def fill(c, b):
        ivec = idx_s[pl.ds(c * CHUNK, LANES)] * HIDDEN
        for r in range(CHUNK):
            off = ivec[r]

            @plsc.parallel_loop(0, NGRP, LBATCH, unroll=2)
            def _(g, r=r, off=off):
                vals = tuple(
                    table_v[pl.ds(pl.multiple_of(
                        off + (g + j) * LANES, LANES), LANES)]
                    for j in range(LBATCH))
                for j in range(LBATCH):
                    bufs[b][pl.ds(r * HIDDEN + (g + j) * LANES, LANES)] = (
                        vals[j])

    def store(c, b):
        return pltpu.make_async_copy(
            bufs[b],
            out_hbm.at[pl.ds((base + c * CHUNK) * HIDDEN, CHUNK * HIDDEN)],
            ssems[b])

    for b in range(NBUF):
        fill(b, b)
        store(b, b).start()

    def step(c, carry):
        for bb in range(NBUF):
            @pl.when(lax.rem(c, NBUF) == bb)
            def _(bb=bb):
                store(c - NBUF, bb).wait()
                fill(c, bb)
                store(c, bb).start()
        return carry

    lax.fori_loop(NBUF, NCHUNK, step, 0)

    for b in range(NBUF):
        store(NCHUNK - NBUF + b, (NCHUNK - NBUF + b) % NBUF).wait()


@jax.jit
def kernel(clause_indices, embeddings):
    idx = clause_indices.astype(jnp.int32)
    table_flat = embeddings.reshape(NUM_CLAUSES_P1 * HIDDEN)
    mesh = plsc.VectorSubcoreMesh(core_axis_name="c", subcore_axis_name="s")
    f = pl.kernel(
        _sc_body,
        out_type=jax.ShapeDtypeStruct((BATCH * HIDDEN,), jnp.float32),
        mesh=mesh,
        compiler_params=pltpu.CompilerParams(needs_layout_passes=False),
        scratch_types=[
            pltpu.VMEM((NUM_CLAUSES_P1 * HIDDEN,), jnp.float32),
            pltpu.VMEM((B_PER_W,), jnp.int32),
            pltpu.VMEM((CHUNK * HIDDEN,), jnp.float32),
            pltpu.VMEM((CHUNK * HIDDEN,), jnp.float32),
            pltpu.SemaphoreType.DMA,
            pltpu.SemaphoreType.DMA,
        ],
    )
    return f(idx, table_flat).reshape(BATCH, HIDDEN)


# speedup vs baseline: 20.2456x; 1.0070x over previous
import jax
import jax.numpy as jnp
from jax import lax
from jax.experimental import pallas as pl
from jax.experimental.pallas import tpu as pltpu
from jax.experimental.pallas import tpu_sc as plsc

NUM_CLAUSES_P1 = 9
HIDDEN = 2048
LANES = 16
NGRP = HIDDEN // LANES
BATCH = 16384

_INFO = plsc.get_sparse_core_info()
NC = _INFO.num_cores
NS = _INFO.num_subcores
NW = NC * NS
B_PER_W = BATCH // NW
CHUNK = 16
NCHUNK = B_PER_W // CHUNK
NBUF = 2
LBATCH = 8


def _sc_body(idx_hbm, table_hbm, out_hbm, table_sh, idx_s,
             buf0, buf1, fs0, fs1, ss0, ss1):
    bufs = (buf0, buf1)
    fsems = (fs0, fs1)
    ssems = (ss0, ss1)
    cid = lax.axis_index("c")
    sid = lax.axis_index("s")
    wid = sid * NC + cid
    base = wid * B_PER_W

    pltpu.sync_copy(idx_hbm.at[pl.ds(base, B_PER_W)], idx_s)

    @pl.when(sid == 0)
    def _():
        pltpu.sync_copy(table_hbm, table_sh)

    plsc.subcore_barrier()

    def fill(c, b):
        ivec = idx_s[pl.ds(c * CHUNK, LANES)] * HIDDEN
        for r in range(CHUNK):
            pltpu.make_async_copy(
                table_sh.at[pl.ds(pl.multiple_of(ivec[r], HIDDEN), HIDDEN)],
                bufs[b].at[pl.ds(r * HIDDEN, HIDDEN)],
                fsems[b],
            ).start()
        for r in range(CHUNK):
            pltpu.make_async_copy(
                table_sh.at[pl.ds(0, HIDDEN)],
                bufs[b].at[pl.ds(r * HIDDEN, HIDDEN)],
                fsems[b],
            ).wait()

    def store(c, b):
        return pltpu.make_async_copy(
            bufs[b],
            out_hbm.at[pl.ds((base + c * CHUNK) * HIDDEN, CHUNK * HIDDEN)],
            ssems[b])

    for b in range(NBUF):
        fill(b, b)
        store(b, b).start()

    def step(c, carry):
        for bb in range(NBUF):
            @pl.when(lax.rem(c, NBUF) == bb)
            def _(bb=bb):
                store(c - NBUF, bb).wait()
                fill(c, bb)
                store(c, bb).start()
        return carry

    lax.fori_loop(NBUF, NCHUNK, step, 0)

    for b in range(NBUF):
        store(NCHUNK - NBUF + b, (NCHUNK - NBUF + b) % NBUF).wait()


@jax.jit
def kernel(clause_indices, embeddings):
    idx = clause_indices.astype(jnp.int32)
    table_flat = embeddings.reshape(NUM_CLAUSES_P1 * HIDDEN)
    mesh = plsc.VectorSubcoreMesh(core_axis_name="c", subcore_axis_name="s")
    f = pl.kernel(
        _sc_body,
        out_type=jax.ShapeDtypeStruct((BATCH * HIDDEN,), jnp.float32),
        mesh=mesh,
        compiler_params=pltpu.CompilerParams(needs_layout_passes=False),
        scratch_types=[
            pltpu.VMEM_SHARED((NUM_CLAUSES_P1 * HIDDEN,), jnp.float32),
            pltpu.VMEM((B_PER_W,), jnp.int32),
            pltpu.VMEM((CHUNK * HIDDEN,), jnp.float32),
            pltpu.VMEM((CHUNK * HIDDEN,), jnp.float32),
            pltpu.SemaphoreType.DMA,
            pltpu.SemaphoreType.DMA,
            pltpu.SemaphoreType.DMA,
            pltpu.SemaphoreType.DMA,
        ],
    )
    return f(idx, table_flat).reshape(BATCH, HIDDEN)


# speedup vs baseline: 23.9957x; 1.1852x over previous
import jax
import jax.numpy as jnp
from jax import lax
from jax.experimental import pallas as pl
from jax.experimental.pallas import tpu as pltpu
from jax.experimental.pallas import tpu_sc as plsc

NUM_CLAUSES_P1 = 9
HIDDEN = 2048
LANES = 16
NGRP = HIDDEN // LANES
BATCH = 16384

_INFO = plsc.get_sparse_core_info()
NC = _INFO.num_cores
NS = _INFO.num_subcores
NW = NC * NS
B_PER_W = BATCH // NW

CHUNK = 8
FILL_CHUNKS = 28
FILL_ROWS = CHUNK * FILL_CHUNKS
SC_GROUP0 = FILL_ROWS // LANES
NGROUPS = B_PER_W // LANES
REP = 16
NDESC_MAX = 32
PAD_ROWS = NW * NUM_CLAUSES_P1
OUT_ROWS = BATCH + PAD_ROWS
NBUF = 2


def _sc_body(idx_hbm, table_hbm, out_hbm, table_v, idx_s, pos_flat, pos2d,
             rep0, rep1, fb0, fb1, rs0, rs1, ss0, ss1):
    reps = (rep0, rep1)
    rsems = (rs0, rs1)
    fbufs = (fb0, fb1)
    ssems = (ss0, ss1)
    cid = lax.axis_index("c")
    sid = lax.axis_index("s")
    wid = sid * NC + cid
    base = wid * B_PER_W
    lane = lax.iota(jnp.int32, LANES)

    pltpu.sync_copy(table_hbm, table_v)
    pltpu.sync_copy(idx_hbm.at[pl.ds(base, B_PER_W)], idx_s)

    seg = []
    cursor = jnp.int32(0)
    full_mask = jnp.ones((LANES,), jnp.bool_)
    for k in range(NUM_CLAUSES_P1):
        start_k = cursor

        def scan_step(rg, cur, k=k):
            ivec = idx_s[pl.ds(rg * LANES, LANES)]
            posv = base + rg * LANES + lane
            m = ivec == k
            plsc.store_compressed(pos_flat.at[pl.ds(cur, LANES)], posv,
                                  mask=m)
            return cur + plsc.all_reduce_population_count(m)[0]

        cursor = lax.fori_loop(SC_GROUP0, NGROUPS, scan_step, cursor)
        dumpv = jnp.zeros((LANES,), jnp.int32) + (
            BATCH + wid * NUM_CLAUSES_P1 + k)
        plsc.store_compressed(pos_flat.at[pl.ds(cursor, LANES)], dumpv,
                              mask=full_mask)
        cursor = ((cursor + LANES - 1) // LANES) * LANES
        seg.append((start_k // LANES, cursor // LANES))

    for d in range(NDESC_MAX):
        pos2d[d] = pos_flat[pl.ds(d * LANES, LANES)]

    def build(k, b):
        def brow(rr, carry):
            @plsc.parallel_loop(0, NGRP, 1, unroll=16)
            def _(g, k=k):
                reps[b][rr, pl.ds(g * LANES, LANES)] = (
                    table_v[pl.ds(k * HIDDEN + g * LANES, LANES)])
            return carry

        lax.fori_loop(0, REP, brow, 0)

    def issue(k, b):
        lo, hi = seg[k]

        def istep(d, carry):
            pltpu.make_async_copy(
                reps[b], out_hbm.at[pos2d.at[d]], rsems[b]).start()
            return carry

        lax.fori_loop(lo, hi, istep, 0)

    def drain(k, b):
        lo, hi = seg[k]

        def wstep(d, carry):
            pltpu.make_async_copy(
                reps[b], out_hbm.at[pos2d.at[0]], rsems[b]).wait()
            return carry

        lax.fori_loop(lo, hi, wstep, 0)

    def fill(c, b):
        ivec = idx_s[pl.ds(c * CHUNK, LANES)] * HIDDEN
        for r in range(CHUNK):
            off = ivec[r]

            @plsc.parallel_loop(0, NGRP, 1, unroll=16)
            def _(g, r=r, off=off):
                fbufs[b][r, pl.ds(g * LANES, LANES)] = (
                    table_v[pl.ds(pl.multiple_of(off + g * LANES, LANES),
                                  LANES)])

    def fstore(c, b):
        return pltpu.make_async_copy(
            fbufs[b],
            out_hbm.at[pl.ds(base + c * CHUNK, CHUNK)],
            ssems[b])

    def fill_step(c, carry):
        for bb in range(NBUF):
            @pl.when(lax.rem(c, NBUF) == bb)
            def _(bb=bb):
                @pl.when(c >= NBUF)
                def _():
                    fstore(c - NBUF, bb).wait()

                fill(c, bb)
                fstore(c, bb).start()

        return carry

    for k in range(2):
        build(k, k)
        issue(k, k)
    for k in range(2, NUM_CLAUSES_P1):
        b = k % 2
        lax.fori_loop(4 * (k - 2), 4 * (k - 1), fill_step, 0)
        drain(k - 2, b)
        build(k, b)
        issue(k, b)
    drain(NUM_CLAUSES_P1 - 2, (NUM_CLAUSES_P1 - 2) % 2)
    drain(NUM_CLAUSES_P1 - 1, (NUM_CLAUSES_P1 - 1) % 2)
    for b in range(NBUF):
        c = FILL_CHUNKS - NBUF + b
        fstore(c, c % NBUF).wait()


@jax.jit
def kernel(clause_indices, embeddings):
    idx = clause_indices.astype(jnp.int32)
    table_flat = embeddings.reshape(NUM_CLAUSES_P1 * HIDDEN)
    mesh = plsc.VectorSubcoreMesh(core_axis_name="c", subcore_axis_name="s")
    f = pl.kernel(
        _sc_body,
        out_type=jax.ShapeDtypeStruct((OUT_ROWS, HIDDEN), jnp.float32),
        mesh=mesh,
        compiler_params=pltpu.CompilerParams(needs_layout_passes=False),
        scratch_types=[
            pltpu.VMEM((NUM_CLAUSES_P1 * HIDDEN,), jnp.float32),
            pltpu.VMEM((B_PER_W,), jnp.int32),
            pltpu.VMEM((NDESC_MAX * LANES,), jnp.int32),
            pltpu.VMEM((NDESC_MAX, LANES), jnp.int32),
            pltpu.VMEM((REP, HIDDEN), jnp.float32),
            pltpu.VMEM((REP, HIDDEN), jnp.float32),
            pltpu.VMEM((CHUNK, HIDDEN), jnp.float32),
            pltpu.VMEM((CHUNK, HIDDEN), jnp.float32),
            pltpu.SemaphoreType.DMA,
            pltpu.SemaphoreType.DMA,
            pltpu.SemaphoreType.DMA,
            pltpu.SemaphoreType.DMA,
        ],
    )
    return f(idx, table_flat)[:BATCH]


# speedup vs baseline: 24.2503x; 1.0106x over previous
import jax
import jax.numpy as jnp
from jax import lax
from jax.experimental import pallas as pl
from jax.experimental.pallas import tpu as pltpu
from jax.experimental.pallas import tpu_sc as plsc

NUM_CLAUSES_P1 = 9
HIDDEN = 2048
LANES = 16
NGRP = HIDDEN // LANES
BATCH = 16384

_INFO = plsc.get_sparse_core_info()
NC = _INFO.num_cores
NS = _INFO.num_subcores
NW = NC * NS
B_PER_W = BATCH // NW

CHUNK = 8
FILL_CHUNKS = 16
FILL_ROWS = CHUNK * FILL_CHUNKS
SC_GROUP0 = FILL_ROWS // LANES
NGROUPS = B_PER_W // LANES
REP = 16
NDESC_MAX = 40
PAD_ROWS = NW * NUM_CLAUSES_P1
OUT_ROWS = BATCH + PAD_ROWS
NBUF = 2


def _sc_body(idx_hbm, table_hbm, out_hbm, table_v, idx_s, pos_flat, pos2d,
             rep0, rep1, fb0, fb1, rs0, rs1, ss0, ss1):
    reps = (rep0, rep1)
    rsems = (rs0, rs1)
    fbufs = (fb0, fb1)
    ssems = (ss0, ss1)
    cid = lax.axis_index("c")
    sid = lax.axis_index("s")
    wid = sid * NC + cid
    base = wid * B_PER_W
    lane = lax.iota(jnp.int32, LANES)

    pltpu.sync_copy(table_hbm, table_v)
    pltpu.sync_copy(idx_hbm.at[pl.ds(base, B_PER_W)], idx_s)

    seg = []
    cursor = jnp.int32(0)
    full_mask = jnp.ones((LANES,), jnp.bool_)
    for k in range(NUM_CLAUSES_P1):
        start_k = cursor

        def scan_step(rg, cur, k=k):
            ivec = idx_s[pl.ds(rg * LANES, LANES)]
            posv = base + rg * LANES + lane
            m = ivec == k
            plsc.store_compressed(pos_flat.at[pl.ds(cur, LANES)], posv,
                                  mask=m)
            return cur + plsc.all_reduce_population_count(m)[0]

        cursor = lax.fori_loop(SC_GROUP0, NGROUPS, scan_step, cursor)
        dumpv = jnp.zeros((LANES,), jnp.int32) + (
            BATCH + wid * NUM_CLAUSES_P1 + k)
        plsc.store_compressed(pos_flat.at[pl.ds(cursor, LANES)], dumpv,
                              mask=full_mask)
        cursor = ((cursor + LANES - 1) // LANES) * LANES
        seg.append((start_k // LANES, cursor // LANES))

    for d in range(NDESC_MAX):
        pos2d[d] = pos_flat[pl.ds(d * LANES, LANES)]

    def build(k, b):
        def brow(rr, carry):
            @plsc.parallel_loop(0, NGRP, 1, unroll=16)
            def _(g, k=k):
                reps[b][rr, pl.ds(g * LANES, LANES)] = (
                    table_v[pl.ds(k * HIDDEN + g * LANES, LANES)])
            return carry

        lax.fori_loop(0, REP, brow, 0)

    def issue(k, b):
        lo, hi = seg[k]

        def istep(d, carry):
            pltpu.make_async_copy(
                reps[b], out_hbm.at[pos2d.at[d]], rsems[b]).start()
            return carry

        lax.fori_loop(lo, hi, istep, 0)

    def drain(k, b):
        lo, hi = seg[k]

        def wstep(d, carry):
            pltpu.make_async_copy(
                reps[b], out_hbm.at[pos2d.at[0]], rsems[b]).wait()
            return carry

        lax.fori_loop(lo, hi, wstep, 0)

    def fill(c, b):
        ivec = idx_s[pl.ds(c * CHUNK, LANES)] * HIDDEN
        for r in range(CHUNK):
            off = ivec[r]

            @plsc.parallel_loop(0, NGRP, 1, unroll=16)
            def _(g, r=r, off=off):
                fbufs[b][r, pl.ds(g * LANES, LANES)] = (
                    table_v[pl.ds(pl.multiple_of(off + g * LANES, LANES),
                                  LANES)])

    def fstore(c, b):
        return pltpu.make_async_copy(
            fbufs[b],
            out_hbm.at[pl.ds(base + c * CHUNK, CHUNK)],
            ssems[b])

    def fill_step(c, carry):
        for bb in range(NBUF):
            @pl.when(lax.rem(c, NBUF) == bb)
            def _(bb=bb):
                @pl.when(c >= NBUF)
                def _():
                    fstore(c - NBUF, bb).wait()

                fill(c, bb)
                fstore(c, bb).start()

        return carry

    for k in range(2):
        build(k, k)
        issue(k, k)
    for k in range(2, NUM_CLAUSES_P1):
        b = k % 2
        lax.fori_loop(2 * (k - 2), 2 * (k - 1), fill_step, 0)
        drain(k - 2, b)
        build(k, b)
        issue(k, b)
    lax.fori_loop(2 * (NUM_CLAUSES_P1 - 2), FILL_CHUNKS, fill_step, 0)
    drain(NUM_CLAUSES_P1 - 2, (NUM_CLAUSES_P1 - 2) % 2)
    drain(NUM_CLAUSES_P1 - 1, (NUM_CLAUSES_P1 - 1) % 2)
    for b in range(NBUF):
        c = FILL_CHUNKS - NBUF + b
        fstore(c, c % NBUF).wait()


@jax.jit
def kernel(clause_indices, embeddings):
    idx = clause_indices.astype(jnp.int32)
    table_flat = embeddings.reshape(NUM_CLAUSES_P1 * HIDDEN)
    mesh = plsc.VectorSubcoreMesh(core_axis_name="c", subcore_axis_name="s")
    f = pl.kernel(
        _sc_body,
        out_type=jax.ShapeDtypeStruct((OUT_ROWS, HIDDEN), jnp.float32),
        mesh=mesh,
        compiler_params=pltpu.CompilerParams(needs_layout_passes=False),
        scratch_types=[
            pltpu.VMEM((NUM_CLAUSES_P1 * HIDDEN,), jnp.float32),
            pltpu.VMEM((B_PER_W,), jnp.int32),
            pltpu.VMEM((NDESC_MAX * LANES,), jnp.int32),
            pltpu.VMEM((NDESC_MAX, LANES), jnp.int32),
            pltpu.VMEM((REP, HIDDEN), jnp.float32),
            pltpu.VMEM((REP, HIDDEN), jnp.float32),
            pltpu.VMEM((CHUNK, HIDDEN), jnp.float32),
            pltpu.VMEM((CHUNK, HIDDEN), jnp.float32),
            pltpu.SemaphoreType.DMA,
            pltpu.SemaphoreType.DMA,
            pltpu.SemaphoreType.DMA,
            pltpu.SemaphoreType.DMA,
        ],
    )
    return f(idx, table_flat)[:BATCH]
